# baseline (speedup 1.0000x reference)
import functools

import jax
import jax.numpy as jnp
from jax import lax
from jax.experimental import pallas as pl
from jax.experimental.pallas import tpu as pltpu
from jax.experimental.pallas import tpu_sc as plsc

E = 20000
C = 1024
P = 32
PPAD = 128
G = PPAD // P
EW = E // G
L = 2048

NC = 2
NS = 16
NW = NC * NS
BPW = L // NW
HPW = BPW // 2

LBLK = 1024
NLB = L // LBLK
RBLK = 2000
NRB = E // RBLK

_NT = (((1,), (1,)), ((), ()))


def _sc_gather(ecw, adj, epww, idx, widx):
    mesh = plsc.VectorSubcoreMesh(core_axis_name="c", subcore_axis_name="s")

    @functools.partial(
        pl.kernel,
        mesh=mesh,
        out_type=(
            jax.ShapeDtypeStruct((L, C), jnp.float32),
            jax.ShapeDtypeStruct((L, C), jnp.float32),
            jax.ShapeDtypeStruct((L, PPAD), jnp.float32),
        ),
        scratch_types=[
            pltpu.VMEM((HPW,), jnp.int32),
            pltpu.VMEM((HPW,), jnp.int32),
            pltpu.VMEM((BPW,), jnp.int32),
            pltpu.VMEM((HPW, C), jnp.float32),
            pltpu.VMEM((HPW, C), jnp.float32),
            pltpu.VMEM((BPW, PPAD), jnp.float32),
            pltpu.SemaphoreType.DMA,
            pltpu.SemaphoreType.DMA,
            pltpu.SemaphoreType.DMA,
            pltpu.SemaphoreType.DMA,
            pltpu.SemaphoreType.DMA,
            pltpu.SemaphoreType.DMA,
        ],
    )
    def k(ecw_hbm, adj_hbm, epww_hbm, idx_hbm, widx_hbm,
          gecw_hbm, gadj_hbm, gepw_hbm,
          ilo_v, ihi_v, widx_v, b0, b1, bw,
          sg0, sg1, sgw, sw0, sw1, sww):
        wid = lax.axis_index("s") * NC + lax.axis_index("c")
        base = wid * BPW
        pltpu.sync_copy(idx_hbm.at[pl.ds(base, HPW)], ilo_v)
        pltpu.sync_copy(idx_hbm.at[pl.ds(base + HPW, HPW)], ihi_v)
        pltpu.sync_copy(widx_hbm.at[pl.ds(base, BPW)], widx_v)
        c0 = pltpu.async_copy(ecw_hbm.at[ilo_v], b0, sg0)
        c1 = pltpu.async_copy(ecw_hbm.at[ihi_v], b1, sg1)
        cw = pltpu.async_copy(epww_hbm.at[widx_v], bw, sgw)
        c0.wait()
        w0 = pltpu.async_copy(b0, gecw_hbm.at[pl.ds(base, HPW)], sw0)
        c1.wait()
        w1 = pltpu.async_copy(b1, gecw_hbm.at[pl.ds(base + HPW, HPW)], sw1)
        w0.wait()
        c2 = pltpu.async_copy(adj_hbm.at[ilo_v], b0, sg0)
        w1.wait()
        c3 = pltpu.async_copy(adj_hbm.at[ihi_v], b1, sg1)
        cw.wait()
        ww = pltpu.async_copy(bw, gepw_hbm.at[pl.ds(base, BPW)], sww)
        c2.wait()
        w2 = pltpu.async_copy(b0, gadj_hbm.at[pl.ds(base, HPW)], sw0)
        c3.wait()
        w3 = pltpu.async_copy(b1, gadj_hbm.at[pl.ds(base + HPW, HPW)], sw1)
        w2.wait()
        w3.wait()
        ww.wait()

    return k(ecw, adj, epww, idx, widx)


def _reduce_body(gecw_ref, gadj_ref, sc_blk_ref, sc_full_ref, gepw_ref,
                 off_ref, ccw_ref, a_ref, bm_ref,
                 accw_ref, accadj_ref, accxw_ref):
    i = pl.program_id(0)

    @pl.when(i == 0)
    def _():
        accw_ref[...] = jnp.zeros_like(accw_ref)
        accadj_ref[...] = jnp.zeros_like(accadj_ref)
        accxw_ref[...] = jnp.zeros_like(accxw_ref)

    adj_blk = gadj_ref[...]
    w = jax.nn.sigmoid(gecw_ref[...]) * adj_blk
    ones = jnp.ones((1, LBLK), jnp.float32)
    accw_ref[...] += jnp.dot(ones, w)
    accadj_ref[...] += jnp.dot(ones, adj_blk)
    accxw_ref[...] += jnp.dot(sc_blk_ref[...], w)

    @pl.when(i == NLB - 1)
    def _():
        mask = accadj_ref[...] > 0.0
        a1 = jnp.where(mask, accxw_ref[...] / accw_ref[...], 0.0)
        ccw = ccw_ref[...]
        mg = jnp.max(ccw, axis=0, keepdims=True)
        ex = jnp.exp(ccw - mg)
        numer = jnp.dot(a1, ex)
        denom = jnp.dot(mask.astype(jnp.float32), ex)
        a_ref[...] = numer / denom

        wide = gepw_ref[...]
        offc = off_ref[...]
        grp = lax.broadcasted_iota(jnp.int32, (L, PPAD), 1) // P
        sel = grp == offc
        m3w = jnp.max(wide, axis=0, keepdims=True)
        m3 = jnp.maximum(
            jnp.maximum(m3w[:, 0:P], m3w[:, P:2 * P]),
            jnp.maximum(m3w[:, 2 * P:3 * P], m3w[:, 3 * P:4 * P]))
        m3b = jnp.concatenate([m3, m3, m3, m3], axis=1)
        e3 = jnp.where(sel, jnp.exp(wide - m3b), 0.0)
        s3w = jnp.sum(e3, axis=0, keepdims=True)
        t3w = jnp.dot(sc_full_ref[...], e3)
        s3 = (s3w[:, 0:P] + s3w[:, P:2 * P]
              + s3w[:, 2 * P:3 * P] + s3w[:, 3 * P:4 * P])
        t3 = (t3w[:, 0:P] + t3w[:, P:2 * P]
              + t3w[:, 2 * P:3 * P] + t3w[:, 3 * P:4 * P])
        bm_ref[...] = t3 / s3


def _tc_reduce(gecw, gadj, gepw, off_col, scores, ccw):
    return pl.pallas_call(
        _reduce_body,
        grid=(NLB,),
        in_specs=[
            pl.BlockSpec((LBLK, C), lambda i: (i, 0)),
            pl.BlockSpec((LBLK, C), lambda i: (i, 0)),
            pl.BlockSpec((1, LBLK), lambda i: (0, i)),
            pl.BlockSpec((1, L), lambda i: (0, 0)),
            pl.BlockSpec((L, PPAD), lambda i: (0, 0)),
            pl.BlockSpec((L, 1), lambda i: (0, 0)),
            pl.BlockSpec((C, C), lambda i: (0, 0)),
        ],
        out_specs=[
            pl.BlockSpec((1, C), lambda i: (0, 0)),
            pl.BlockSpec((1, P), lambda i: (0, 0)),
        ],
        out_shape=[
            jax.ShapeDtypeStruct((1, C), jnp.float32),
            jax.ShapeDtypeStruct((1, P), jnp.float32),
        ],
        scratch_shapes=[
            pltpu.VMEM((1, C), jnp.float32),
            pltpu.VMEM((1, C), jnp.float32),
            pltpu.VMEM((1, C), jnp.float32),
        ],
        compiler_params=pltpu.CompilerParams(
            dimension_semantics=("arbitrary",)),
    )(gecw, gadj, scores, scores, gepw, off_col, ccw)


def _stream_body(ecw_ref, adj_ref, epw_ref, lam_ref, gue_ref, sli_ref,
                 a_ref, bm_ref, y_ref):
    adj_blk = adj_ref[...]
    w = jax.nn.sigmoid(ecw_ref[...]) * adj_blk
    ones = jnp.ones((1, C), jnp.float32)
    s = lax.dot_general(ones, w, _NT)
    num = lax.dot_general(a_ref[...], w, _NT)
    ya = num / s
    d = epw_ref[...]
    e3 = jnp.exp(d - jnp.max(d, axis=1, keepdims=True))
    d2n = e3 / jnp.sum(e3, axis=1, keepdims=True)
    yb = lax.dot_general(bm_ref[...], d2n, _NT)
    lam = jax.nn.sigmoid(lam_ref[0])
    gue = jax.nn.sigmoid(gue_ref[0])
    sli = jax.nn.sigmoid(sli_ref[0])
    y_ = (1.0 - lam) * ya + lam * yb
    y_ = jnp.clip(y_, 1e-8, 1.0 - 1e-8)
    y_ref[0] = (1.0 - sli) * y_ + gue * (1.0 - y_)


def _tc_stream(ecw, adj, epw, lam3, gue3, sli3, a, bm):
    return pl.pallas_call(
        _stream_body,
        grid=(NRB,),
        in_specs=[
            pl.BlockSpec((RBLK, C), lambda i: (i, 0)),
            pl.BlockSpec((RBLK, C), lambda i: (i, 0)),
            pl.BlockSpec((RBLK, P), lambda i: (i, 0)),
            pl.BlockSpec((1, 1, RBLK), lambda i: (i, 0, 0)),
            pl.BlockSpec((1, 1, RBLK), lambda i: (i, 0, 0)),
            pl.BlockSpec((1, 1, RBLK), lambda i: (i, 0, 0)),
            pl.BlockSpec((1, C), lambda i: (0, 0)),
            pl.BlockSpec((1, P), lambda i: (0, 0)),
        ],
        out_specs=pl.BlockSpec((1, 1, RBLK), lambda i: (i, 0, 0)),
        out_shape=jax.ShapeDtypeStruct((NRB, 1, RBLK), jnp.float32),
        compiler_params=pltpu.CompilerParams(
            dimension_semantics=("arbitrary",)),
    )(ecw, adj, epw, lam3, gue3, sli3, a, bm)


def kernel(exer_list, score_list, school_feature, exer_conc_adj,
           school_feature_dim_w, exer_conc_w, conc_conc_w, exer_pote_w,
           lambd, guess, slide):
    del school_feature, school_feature_dim_w
    idx = exer_list.reshape(L).astype(jnp.int32)
    widx = idx // G
    off_col = (idx % G).reshape(L, 1)
    scores = score_list.reshape(1, L).astype(jnp.float32)
    epww = exer_pote_w.reshape(EW, PPAD)
    gecw, gadj, gepw = _sc_gather(exer_conc_w, exer_conc_adj, epww,
                                  idx, widx)
    a, bm = _tc_reduce(gecw, gadj, gepw, off_col, scores, conc_conc_w)
    lam3 = lambd.reshape(NRB, 1, RBLK)
    gue3 = guess.reshape(NRB, 1, RBLK)
    sli3 = slide.reshape(NRB, 1, RBLK)
    y3 = _tc_stream(exer_conc_w, exer_conc_adj, exer_pote_w,
                    lam3, gue3, sli3, a, bm)
    return (a, y3.reshape(1, E))
